# baseline (speedup 1.0000x reference)
import functools

import jax
import jax.numpy as jnp
from jax import lax
from jax.experimental import pallas as pl
from jax.experimental.pallas import tpu as pltpu
from jax.experimental.pallas import tpu_sc as plsc

BATCH = 16384
DIM = 64
DIM_MLP = 256
CHUNK = 64


def _make_gather(d):
    info = plsc.get_sparse_core_info()
    nc, ns = info.num_cores, info.num_subcores
    nw = nc * ns
    b_per_w = BATCH // nw
    n_chunks = b_per_w // CHUNK
    mesh = plsc.VectorSubcoreMesh(core_axis_name="c", subcore_axis_name="s")
    f32 = jnp.float32

    @functools.partial(
        pl.kernel,
        mesh=mesh,
        out_type=[
            jax.ShapeDtypeStruct((BATCH, d), f32),
            jax.ShapeDtypeStruct((BATCH, d), f32),
        ],
        scratch_types=[
            pltpu.VMEM((n_chunks, CHUNK), jnp.int32),
            pltpu.VMEM((n_chunks, CHUNK), jnp.int32),
            pltpu.VMEM((CHUNK, d), f32),
            pltpu.VMEM((CHUNK, d), f32),
            pltpu.VMEM((CHUNK, d), f32),
            pltpu.VMEM((CHUNK, d), f32),
            pltpu.SemaphoreType.DMA,
            pltpu.SemaphoreType.DMA,
            pltpu.SemaphoreType.DMA,
            pltpu.SemaphoreType.DMA,
        ],
    )
    def gather_kernel(u_idx_hbm, s_idx_hbm, ut_hbm, it_hbm,
                      out_u, out_i,
                      idx_u, idx_i, bu0, bu1, bi0, bi1,
                      su0, su1, si0, si1):
        wid = lax.axis_index("s") * nc + lax.axis_index("c")
        base = wid * b_per_w
        bufs_u, bufs_i = (bu0, bu1), (bi0, bi1)
        sems_u, sems_i = (su0, su1), (si0, si1)
        for j in range(n_chunks):
            pltpu.sync_copy(u_idx_hbm.at[pl.ds(base + j * CHUNK, CHUNK)],
                            idx_u.at[j])
            pltpu.sync_copy(s_idx_hbm.at[pl.ds(base + j * CHUNK, CHUNK)],
                            idx_i.at[j])

        cps = [None, None]
        cps[0] = (
            pltpu.async_copy(ut_hbm.at[idx_u.at[0]], bufs_u[0], sems_u[0]),
            pltpu.async_copy(it_hbm.at[idx_i.at[0]], bufs_i[0], sems_i[0]),
        )
        for j in range(n_chunks):
            s = j % 2
            n = (j + 1) % 2
            if j + 1 < n_chunks:
                cps[n] = (
                    pltpu.async_copy(ut_hbm.at[idx_u.at[j + 1]],
                                     bufs_u[n], sems_u[n]),
                    pltpu.async_copy(it_hbm.at[idx_i.at[j + 1]],
                                     bufs_i[n], sems_i[n]),
                )
            sl = pl.ds(base + j * CHUNK, CHUNK)
            cps[s][0].wait()
            pltpu.sync_copy(bufs_u[s], out_u.at[sl])
            cps[s][1].wait()
            pltpu.sync_copy(bufs_i[s], out_i.at[sl])

    return gather_kernel


_gather_mlp = _make_gather(DIM_MLP)
_gather_gmf = _make_gather(2 * DIM)


def _tpad_body(tt, out):
    t = jnp.transpose(tt[...], (1, 0))
    out[...] = jnp.pad(t, ((0, 0), (0, DIM)))


def _transpose_pad(tab_t):
    n = tab_t.shape[1]
    c = 512
    grid = (-(-n // c),)
    return pl.pallas_call(
        _tpad_body,
        grid=grid,
        in_specs=[pl.BlockSpec((DIM, c), lambda i: (0, i))],
        out_specs=pl.BlockSpec((c, 2 * DIM), lambda i: (i, 0)),
        out_shape=jax.ShapeDtypeStruct((n, 2 * DIM), jnp.float32),
    )(tab_t)


def _bdot(a, b):
    return jnp.dot(a.astype(jnp.bfloat16), b.astype(jnp.bfloat16),
                   preferred_element_type=jnp.float32)


def _dense_body(um, im, w0u, w0i, b0, w1, b1, w2, b2, wpx, bp, out):
    x = _bdot(um[...], w0u[...]) + _bdot(im[...], w0i[...])
    x = jnp.maximum(x + b0[...], 0.0)
    x = jnp.maximum(_bdot(x, w1[...]) + b1[...], 0.0)
    x = jnp.maximum(_bdot(x, w2[...]) + b2[...], 0.0)
    out[...] = _bdot(x, wpx[...]) + bp[...]


def _final_body(ug, ig, xd, wpg, out):
    g = ug[:, :DIM] * ig[:, :DIM]
    out[...] = _bdot(g, wpg[...]) + xd[...]


_T = 1024


def _run_dense(um, im, W0, b0, W1, b1, W2, b2, Wp, bp):
    grid = (BATCH // _T,)
    f32 = jnp.float32
    w0u = W0[:, :DIM_MLP].T
    w0i = W0[:, DIM_MLP:].T
    w1 = W1.T
    w2 = W2.T
    wpx = Wp[:, DIM:].T
    b0r = b0.reshape(1, -1)
    b1r = b1.reshape(1, -1)
    b2r = b2.reshape(1, -1)
    bpr = bp.reshape(1, 1)

    batch_spec = lambda d: pl.BlockSpec((_T, d), lambda i: (i, 0))
    full_spec = lambda a, b: pl.BlockSpec((a, b), lambda i: (0, 0))

    return pl.pallas_call(
        _dense_body,
        grid=grid,
        in_specs=[
            batch_spec(DIM_MLP), batch_spec(DIM_MLP),
            full_spec(256, 256), full_spec(256, 256), full_spec(1, 256),
            full_spec(256, 128), full_spec(1, 128),
            full_spec(128, 64), full_spec(1, 64),
            full_spec(64, 1), full_spec(1, 1),
        ],
        out_specs=pl.BlockSpec((_T, 1), lambda i: (i, 0)),
        out_shape=jax.ShapeDtypeStruct((BATCH, 1), f32),
    )(um, im, w0u, w0i, b0r, w1, b1r, w2, b2r, wpx, bpr)


def _run_final(ug2, ig2, xdot, Wp):
    grid = (BATCH // _T,)
    f32 = jnp.float32
    wpg = Wp[:, :DIM].T
    batch_spec = lambda d: pl.BlockSpec((_T, d), lambda i: (i, 0))
    out = pl.pallas_call(
        _final_body,
        grid=grid,
        in_specs=[
            batch_spec(2 * DIM), batch_spec(2 * DIM), batch_spec(1),
            pl.BlockSpec((DIM, 1), lambda i: (0, 0)),
        ],
        out_specs=pl.BlockSpec((_T, 1), lambda i: (i, 0)),
        out_shape=jax.ShapeDtypeStruct((BATCH, 1), f32),
    )(ug2, ig2, xdot, wpg)
    return out.reshape(-1)


def kernel(userIdx, servIdx, U_gmf, U_mlp, I_gmf, I_mlp, W0, b0, W1, b1, W2, b2, Wp, bp):
    ui = userIdx.astype(jnp.int32)
    si = servIdx.astype(jnp.int32)
    um, im = _gather_mlp(ui, si, U_mlp, I_mlp)
    ugp = _transpose_pad(U_gmf.T)
    igp = _transpose_pad(I_gmf.T)
    ug2, ig2 = _gather_gmf(ui, si, ugp, igp)
    xdot = _run_dense(um, im, W0, b0, W1, b1, W2, b2, Wp, bp)
    return _run_final(ug2, ig2, xdot, Wp)


# speedup vs baseline: 1.7707x; 1.7707x over previous
import functools

import jax
import jax.numpy as jnp
from jax import lax
from jax.experimental import pallas as pl
from jax.experimental.pallas import tpu as pltpu
from jax.experimental.pallas import tpu_sc as plsc

BATCH = 16384
DIM = 64
DIM_MLP = 256
CHUNK = 64


def _make_gather(d):
    info = plsc.get_sparse_core_info()
    nc, ns = info.num_cores, info.num_subcores
    nw = nc * ns
    b_per_w = BATCH // nw
    n_chunks = b_per_w // CHUNK
    mesh = plsc.VectorSubcoreMesh(core_axis_name="c", subcore_axis_name="s")
    f32 = jnp.float32

    @functools.partial(
        pl.kernel,
        mesh=mesh,
        out_type=[
            jax.ShapeDtypeStruct((BATCH, d), f32),
            jax.ShapeDtypeStruct((BATCH, d), f32),
        ],
        scratch_types=[
            pltpu.VMEM((n_chunks, CHUNK), jnp.int32),
            pltpu.VMEM((n_chunks, CHUNK), jnp.int32),
            pltpu.VMEM((CHUNK, d), f32),
            pltpu.VMEM((CHUNK, d), f32),
            pltpu.VMEM((CHUNK, d), f32),
            pltpu.VMEM((CHUNK, d), f32),
            pltpu.SemaphoreType.DMA,
            pltpu.SemaphoreType.DMA,
            pltpu.SemaphoreType.DMA,
            pltpu.SemaphoreType.DMA,
        ],
    )
    def gather_kernel(u_idx_hbm, s_idx_hbm, ut_hbm, it_hbm,
                      out_u, out_i,
                      idx_u, idx_i, bu0, bu1, bi0, bi1,
                      su0, su1, si0, si1):
        wid = lax.axis_index("s") * nc + lax.axis_index("c")
        base = wid * b_per_w
        bufs_u, bufs_i = (bu0, bu1), (bi0, bi1)
        sems_u, sems_i = (su0, su1), (si0, si1)
        for j in range(n_chunks):
            pltpu.sync_copy(u_idx_hbm.at[pl.ds(base + j * CHUNK, CHUNK)],
                            idx_u.at[j])
            pltpu.sync_copy(s_idx_hbm.at[pl.ds(base + j * CHUNK, CHUNK)],
                            idx_i.at[j])

        cps = [None, None]
        cps[0] = (
            pltpu.async_copy(ut_hbm.at[idx_u.at[0]], bufs_u[0], sems_u[0]),
            pltpu.async_copy(it_hbm.at[idx_i.at[0]], bufs_i[0], sems_i[0]),
        )
        for j in range(n_chunks):
            s = j % 2
            n = (j + 1) % 2
            if j + 1 < n_chunks:
                cps[n] = (
                    pltpu.async_copy(ut_hbm.at[idx_u.at[j + 1]],
                                     bufs_u[n], sems_u[n]),
                    pltpu.async_copy(it_hbm.at[idx_i.at[j + 1]],
                                     bufs_i[n], sems_i[n]),
                )
            sl = pl.ds(base + j * CHUNK, CHUNK)
            cps[s][0].wait()
            pltpu.sync_copy(bufs_u[s], out_u.at[sl])
            cps[s][1].wait()
            pltpu.sync_copy(bufs_i[s], out_i.at[sl])

    return gather_kernel


_gather_mlp = _make_gather(DIM_MLP)
_gather_gmf = _make_gather(2 * DIM)


def _tpad_body(tt, eye, out):
    t = jax.lax.dot_general(tt[...], eye[...], (((0,), (0,)), ((), ())),
                            preferred_element_type=jnp.float32)
    out[...] = jnp.pad(t, ((0, 0), (0, DIM)))


def _transpose_pad(tab_t):
    n = tab_t.shape[1]
    c = 2048
    grid = (-(-n // c),)
    eye = jnp.eye(DIM, dtype=jnp.float32)
    return pl.pallas_call(
        _tpad_body,
        grid=grid,
        in_specs=[pl.BlockSpec((DIM, c), lambda i: (0, i)),
                  pl.BlockSpec((DIM, DIM), lambda i: (0, 0))],
        out_specs=pl.BlockSpec((c, 2 * DIM), lambda i: (i, 0)),
        out_shape=jax.ShapeDtypeStruct((n, 2 * DIM), jnp.float32),
    )(tab_t, eye)


def _bdot(a, b):
    return jnp.dot(a.astype(jnp.bfloat16), b.astype(jnp.bfloat16),
                   preferred_element_type=jnp.float32)


def _dense_body(um, im, w0u, w0i, b0, w1, b1, w2, b2, wpx, bp, out):
    x = _bdot(um[...], w0u[...]) + _bdot(im[...], w0i[...])
    x = jnp.maximum(x + b0[...], 0.0)
    x = jnp.maximum(_bdot(x, w1[...]) + b1[...], 0.0)
    x = jnp.maximum(_bdot(x, w2[...]) + b2[...], 0.0)
    out[...] = _bdot(x, wpx[...]) + bp[...]


def _final_body(ug, ig, xd, wpg, out):
    g = ug[:, :DIM] * ig[:, :DIM]
    out[...] = _bdot(g, wpg[...]) + xd[...]


_T = 1024


def _run_dense(um, im, W0, b0, W1, b1, W2, b2, Wp, bp):
    grid = (BATCH // _T,)
    f32 = jnp.float32
    w0u = W0[:, :DIM_MLP].T
    w0i = W0[:, DIM_MLP:].T
    w1 = W1.T
    w2 = W2.T
    wpx = Wp[:, DIM:].T
    b0r = b0.reshape(1, -1)
    b1r = b1.reshape(1, -1)
    b2r = b2.reshape(1, -1)
    bpr = bp.reshape(1, 1)

    batch_spec = lambda d: pl.BlockSpec((_T, d), lambda i: (i, 0))
    full_spec = lambda a, b: pl.BlockSpec((a, b), lambda i: (0, 0))

    return pl.pallas_call(
        _dense_body,
        grid=grid,
        in_specs=[
            batch_spec(DIM_MLP), batch_spec(DIM_MLP),
            full_spec(256, 256), full_spec(256, 256), full_spec(1, 256),
            full_spec(256, 128), full_spec(1, 128),
            full_spec(128, 64), full_spec(1, 64),
            full_spec(64, 1), full_spec(1, 1),
        ],
        out_specs=pl.BlockSpec((_T, 1), lambda i: (i, 0)),
        out_shape=jax.ShapeDtypeStruct((BATCH, 1), f32),
    )(um, im, w0u, w0i, b0r, w1, b1r, w2, b2r, wpx, bpr)


def _run_final(ug2, ig2, xdot, Wp):
    grid = (BATCH // _T,)
    f32 = jnp.float32
    wpg = Wp[:, :DIM].T
    batch_spec = lambda d: pl.BlockSpec((_T, d), lambda i: (i, 0))
    out = pl.pallas_call(
        _final_body,
        grid=grid,
        in_specs=[
            batch_spec(2 * DIM), batch_spec(2 * DIM), batch_spec(1),
            pl.BlockSpec((DIM, 1), lambda i: (0, 0)),
        ],
        out_specs=pl.BlockSpec((_T, 1), lambda i: (i, 0)),
        out_shape=jax.ShapeDtypeStruct((BATCH, 1), f32),
    )(ug2, ig2, xdot, wpg)
    return out.reshape(-1)


def kernel(userIdx, servIdx, U_gmf, U_mlp, I_gmf, I_mlp, W0, b0, W1, b1, W2, b2, Wp, bp):
    ui = userIdx.astype(jnp.int32)
    si = servIdx.astype(jnp.int32)
    um, im = _gather_mlp(ui, si, U_mlp, I_mlp)
    ugp = _transpose_pad(U_gmf.T)
    igp = _transpose_pad(I_gmf.T)
    ug2, ig2 = _gather_gmf(ui, si, ugp, igp)
    xdot = _run_dense(um, im, W0, b0, W1, b1, W2, b2, Wp, bp)
    return _run_final(ug2, ig2, xdot, Wp)


# speedup vs baseline: 2.0285x; 1.1456x over previous
import functools

import jax
import jax.numpy as jnp
from jax import lax
from jax.experimental import pallas as pl
from jax.experimental.pallas import tpu as pltpu
from jax.experimental.pallas import tpu_sc as plsc

BATCH = 16384
DIM = 64
DIM_MLP = 256
CHUNK = 64


def _make_gather1(d):
    info = plsc.get_sparse_core_info()
    nc, ns = info.num_cores, info.num_subcores
    nw = nc * ns
    b_per_w = BATCH // nw
    n_chunks = b_per_w // CHUNK
    mesh = plsc.VectorSubcoreMesh(core_axis_name="c", subcore_axis_name="s")
    f32 = jnp.float32

    @functools.partial(
        pl.kernel,
        mesh=mesh,
        out_type=jax.ShapeDtypeStruct((BATCH, d), f32),
        scratch_types=[
            pltpu.VMEM((n_chunks, CHUNK), jnp.int32),
            pltpu.VMEM((CHUNK, d), f32),
            pltpu.VMEM((CHUNK, d), f32),
            pltpu.SemaphoreType.DMA,
            pltpu.SemaphoreType.DMA,
        ],
    )
    def gather1(idx_hbm, tab_hbm, out, idx, b0, b1, s0, s1):
        wid = lax.axis_index("s") * nc + lax.axis_index("c")
        base = wid * b_per_w
        bufs, sems = (b0, b1), (s0, s1)
        for j in range(n_chunks):
            pltpu.sync_copy(idx_hbm.at[pl.ds(base + j * CHUNK, CHUNK)],
                            idx.at[j])
        cps = [None, None]
        cps[0] = pltpu.async_copy(tab_hbm.at[idx.at[0]], bufs[0], sems[0])
        for j in range(n_chunks):
            s = j % 2
            n = (j + 1) % 2
            if j + 1 < n_chunks:
                cps[n] = pltpu.async_copy(tab_hbm.at[idx.at[j + 1]],
                                          bufs[n], sems[n])
            cps[s].wait()
            pltpu.sync_copy(bufs[s], out.at[pl.ds(base + j * CHUNK, CHUNK)])

    return gather1


def _make_gather(d):
    info = plsc.get_sparse_core_info()
    nc, ns = info.num_cores, info.num_subcores
    nw = nc * ns
    b_per_w = BATCH // nw
    n_chunks = b_per_w // CHUNK
    mesh = plsc.VectorSubcoreMesh(core_axis_name="c", subcore_axis_name="s")
    f32 = jnp.float32

    @functools.partial(
        pl.kernel,
        mesh=mesh,
        out_type=[
            jax.ShapeDtypeStruct((BATCH, d), f32),
            jax.ShapeDtypeStruct((BATCH, d), f32),
        ],
        scratch_types=[
            pltpu.VMEM((n_chunks, CHUNK), jnp.int32),
            pltpu.VMEM((n_chunks, CHUNK), jnp.int32),
            pltpu.VMEM((CHUNK, d), f32),
            pltpu.VMEM((CHUNK, d), f32),
            pltpu.VMEM((CHUNK, d), f32),
            pltpu.VMEM((CHUNK, d), f32),
            pltpu.SemaphoreType.DMA,
            pltpu.SemaphoreType.DMA,
            pltpu.SemaphoreType.DMA,
            pltpu.SemaphoreType.DMA,
        ],
    )
    def gather_kernel(u_idx_hbm, s_idx_hbm, ut_hbm, it_hbm,
                      out_u, out_i,
                      idx_u, idx_i, bu0, bu1, bi0, bi1,
                      su0, su1, si0, si1):
        wid = lax.axis_index("s") * nc + lax.axis_index("c")
        base = wid * b_per_w
        bufs_u, bufs_i = (bu0, bu1), (bi0, bi1)
        sems_u, sems_i = (su0, su1), (si0, si1)
        for j in range(n_chunks):
            pltpu.sync_copy(u_idx_hbm.at[pl.ds(base + j * CHUNK, CHUNK)],
                            idx_u.at[j])
            pltpu.sync_copy(s_idx_hbm.at[pl.ds(base + j * CHUNK, CHUNK)],
                            idx_i.at[j])

        cps = [None, None]
        cps[0] = (
            pltpu.async_copy(ut_hbm.at[idx_u.at[0]], bufs_u[0], sems_u[0]),
            pltpu.async_copy(it_hbm.at[idx_i.at[0]], bufs_i[0], sems_i[0]),
        )
        for j in range(n_chunks):
            s = j % 2
            n = (j + 1) % 2
            if j + 1 < n_chunks:
                cps[n] = (
                    pltpu.async_copy(ut_hbm.at[idx_u.at[j + 1]],
                                     bufs_u[n], sems_u[n]),
                    pltpu.async_copy(it_hbm.at[idx_i.at[j + 1]],
                                     bufs_i[n], sems_i[n]),
                )
            sl = pl.ds(base + j * CHUNK, CHUNK)
            cps[s][0].wait()
            pltpu.sync_copy(bufs_u[s], out_u.at[sl])
            cps[s][1].wait()
            pltpu.sync_copy(bufs_i[s], out_i.at[sl])

    return gather_kernel


_gather_mlp = _make_gather(DIM_MLP)
_gather_gmf1 = _make_gather1(2 * DIM)


def _tpad_body(tt, eye, out):
    t = jax.lax.dot_general(tt[...], eye[...], (((0,), (0,)), ((), ())),
                            preferred_element_type=jnp.float32)
    out[...] = jnp.pad(t, ((0, 0), (0, DIM)))


def _transpose_pad(tab_t):
    n = tab_t.shape[1]
    c = 4096
    grid = (-(-n // c),)
    eye = jnp.eye(DIM, dtype=jnp.float32)
    return pl.pallas_call(
        _tpad_body,
        grid=grid,
        in_specs=[pl.BlockSpec((DIM, c), lambda i: (0, i)),
                  pl.BlockSpec((DIM, DIM), lambda i: (0, 0))],
        out_specs=pl.BlockSpec((c, 2 * DIM), lambda i: (i, 0)),
        out_shape=jax.ShapeDtypeStruct((n, 2 * DIM), jnp.float32),
    )(tab_t, eye)


def _bdot(a, b):
    return jnp.dot(a.astype(jnp.bfloat16), b.astype(jnp.bfloat16),
                   preferred_element_type=jnp.float32)


def _dense_body(um, im, w0u, w0i, b0, w1, b1, w2, b2, wpx, bp, out):
    x = _bdot(um[...], w0u[...]) + _bdot(im[...], w0i[...])
    x = jnp.maximum(x + b0[...], 0.0)
    x = jnp.maximum(_bdot(x, w1[...]) + b1[...], 0.0)
    x = jnp.maximum(_bdot(x, w2[...]) + b2[...], 0.0)
    out[...] = _bdot(x, wpx[...]) + bp[...]


def _final_body(ug, ig, xd, wpg, out):
    g = ug[:, :DIM] * ig[:, :DIM]
    out[...] = _bdot(g, wpg[...]) + xd[...]


_T = 1024


def _run_dense(um, im, W0, b0, W1, b1, W2, b2, Wp, bp):
    grid = (BATCH // _T,)
    f32 = jnp.float32
    w0u = W0[:, :DIM_MLP].T
    w0i = W0[:, DIM_MLP:].T
    w1 = W1.T
    w2 = W2.T
    wpx = Wp[:, DIM:].T
    b0r = b0.reshape(1, -1)
    b1r = b1.reshape(1, -1)
    b2r = b2.reshape(1, -1)
    bpr = bp.reshape(1, 1)

    batch_spec = lambda d: pl.BlockSpec((_T, d), lambda i: (i, 0))
    full_spec = lambda a, b: pl.BlockSpec((a, b), lambda i: (0, 0))

    return pl.pallas_call(
        _dense_body,
        grid=grid,
        in_specs=[
            batch_spec(DIM_MLP), batch_spec(DIM_MLP),
            full_spec(256, 256), full_spec(256, 256), full_spec(1, 256),
            full_spec(256, 128), full_spec(1, 128),
            full_spec(128, 64), full_spec(1, 64),
            full_spec(64, 1), full_spec(1, 1),
        ],
        out_specs=pl.BlockSpec((_T, 1), lambda i: (i, 0)),
        out_shape=jax.ShapeDtypeStruct((BATCH, 1), f32),
    )(um, im, w0u, w0i, b0r, w1, b1r, w2, b2r, wpx, bpr)


def _run_final(ug2, ig2, xdot, Wp):
    grid = (BATCH // _T,)
    f32 = jnp.float32
    wpg = Wp[:, :DIM].T
    batch_spec = lambda d: pl.BlockSpec((_T, d), lambda i: (i, 0))
    out = pl.pallas_call(
        _final_body,
        grid=grid,
        in_specs=[
            batch_spec(2 * DIM), batch_spec(2 * DIM), batch_spec(1),
            pl.BlockSpec((DIM, 1), lambda i: (0, 0)),
        ],
        out_specs=pl.BlockSpec((_T, 1), lambda i: (i, 0)),
        out_shape=jax.ShapeDtypeStruct((BATCH, 1), f32),
    )(ug2, ig2, xdot, wpg)
    return out.reshape(-1)


def kernel(userIdx, servIdx, U_gmf, U_mlp, I_gmf, I_mlp, W0, b0, W1, b1, W2, b2, Wp, bp):
    ui = userIdx.astype(jnp.int32)
    si = servIdx.astype(jnp.int32)
    um, im = _gather_mlp(ui, si, U_mlp, I_mlp)
    igp = _transpose_pad(I_gmf.T)
    ig2 = _gather_gmf1(si, igp)
    ugp = _transpose_pad(U_gmf.T)
    ug2 = _gather_gmf1(ui, ugp)
    xdot = _run_dense(um, im, W0, b0, W1, b1, W2, b2, Wp, bp)
    return _run_final(ug2, ig2, xdot, Wp)


# speedup vs baseline: 2.2837x; 1.1258x over previous
import functools

import jax
import jax.numpy as jnp
from jax import lax
from jax.experimental import pallas as pl
from jax.experimental.pallas import tpu as pltpu
from jax.experimental.pallas import tpu_sc as plsc

BATCH = 16384
DIM = 64
DIM_MLP = 256
CHUNK = 64


def _make_gather1(d):
    info = plsc.get_sparse_core_info()
    nc, ns = info.num_cores, info.num_subcores
    nw = nc * ns
    b_per_w = BATCH // nw
    n_chunks = b_per_w // CHUNK
    mesh = plsc.VectorSubcoreMesh(core_axis_name="c", subcore_axis_name="s")
    f32 = jnp.float32

    @functools.partial(
        pl.kernel,
        mesh=mesh,
        out_type=jax.ShapeDtypeStruct((BATCH, d), f32),
        scratch_types=[
            pltpu.VMEM((n_chunks, CHUNK), jnp.int32),
            pltpu.VMEM((CHUNK, d), f32),
            pltpu.VMEM((CHUNK, d), f32),
            pltpu.SemaphoreType.DMA,
            pltpu.SemaphoreType.DMA,
        ],
    )
    def gather1(idx_hbm, tab_hbm, out, idx, b0, b1, s0, s1):
        wid = lax.axis_index("s") * nc + lax.axis_index("c")
        base = wid * b_per_w
        bufs, sems = (b0, b1), (s0, s1)
        for j in range(n_chunks):
            pltpu.sync_copy(idx_hbm.at[pl.ds(base + j * CHUNK, CHUNK)],
                            idx.at[j])
        cps = [None, None]
        cps[0] = pltpu.async_copy(tab_hbm.at[idx.at[0]], bufs[0], sems[0])
        for j in range(n_chunks):
            s = j % 2
            n = (j + 1) % 2
            if j + 1 < n_chunks:
                cps[n] = pltpu.async_copy(tab_hbm.at[idx.at[j + 1]],
                                          bufs[n], sems[n])
            cps[s].wait()
            pltpu.sync_copy(bufs[s], out.at[pl.ds(base + j * CHUNK, CHUNK)])

    return gather1


def _make_gather(d):
    info = plsc.get_sparse_core_info()
    nc, ns = info.num_cores, info.num_subcores
    nw = nc * ns
    b_per_w = BATCH // nw
    n_chunks = b_per_w // CHUNK
    mesh = plsc.VectorSubcoreMesh(core_axis_name="c", subcore_axis_name="s")
    f32 = jnp.float32

    @functools.partial(
        pl.kernel,
        mesh=mesh,
        out_type=[
            jax.ShapeDtypeStruct((BATCH, d), f32),
            jax.ShapeDtypeStruct((BATCH, d), f32),
        ],
        scratch_types=[
            pltpu.VMEM((n_chunks, CHUNK), jnp.int32),
            pltpu.VMEM((n_chunks, CHUNK), jnp.int32),
            pltpu.VMEM((CHUNK, d), f32),
            pltpu.VMEM((CHUNK, d), f32),
            pltpu.VMEM((CHUNK, d), f32),
            pltpu.VMEM((CHUNK, d), f32),
            pltpu.SemaphoreType.DMA,
            pltpu.SemaphoreType.DMA,
            pltpu.SemaphoreType.DMA,
            pltpu.SemaphoreType.DMA,
        ],
    )
    def gather_kernel(u_idx_hbm, s_idx_hbm, ut_hbm, it_hbm,
                      out_u, out_i,
                      idx_u, idx_i, bu0, bu1, bi0, bi1,
                      su0, su1, si0, si1):
        wid = lax.axis_index("s") * nc + lax.axis_index("c")
        base = wid * b_per_w
        bufs_u, bufs_i = (bu0, bu1), (bi0, bi1)
        sems_u, sems_i = (su0, su1), (si0, si1)
        for j in range(n_chunks):
            pltpu.sync_copy(u_idx_hbm.at[pl.ds(base + j * CHUNK, CHUNK)],
                            idx_u.at[j])
            pltpu.sync_copy(s_idx_hbm.at[pl.ds(base + j * CHUNK, CHUNK)],
                            idx_i.at[j])

        cps = [None, None]
        cps[0] = (
            pltpu.async_copy(ut_hbm.at[idx_u.at[0]], bufs_u[0], sems_u[0]),
            pltpu.async_copy(it_hbm.at[idx_i.at[0]], bufs_i[0], sems_i[0]),
        )
        for j in range(n_chunks):
            s = j % 2
            n = (j + 1) % 2
            if j + 1 < n_chunks:
                cps[n] = (
                    pltpu.async_copy(ut_hbm.at[idx_u.at[j + 1]],
                                     bufs_u[n], sems_u[n]),
                    pltpu.async_copy(it_hbm.at[idx_i.at[j + 1]],
                                     bufs_i[n], sems_i[n]),
                )
            sl = pl.ds(base + j * CHUNK, CHUNK)
            cps[s][0].wait()
            pltpu.sync_copy(bufs_u[s], out_u.at[sl])
            cps[s][1].wait()
            pltpu.sync_copy(bufs_i[s], out_i.at[sl])

    return gather_kernel


_gather_mlp = _make_gather(DIM_MLP)
_gather_gmf1 = _make_gather1(2 * DIM)


def _tpad_body(tt, eye, out):
    t = jax.lax.dot_general(tt[...], eye[...], (((0,), (0,)), ((), ())),
                            preferred_element_type=jnp.float32)
    out[...] = jnp.pad(t, ((0, 0), (0, DIM)))


def _transpose_pad(tab_t):
    n = tab_t.shape[1]
    c = 8192
    grid = (-(-n // c),)
    eye = jnp.eye(DIM, dtype=jnp.float32)
    return pl.pallas_call(
        _tpad_body,
        grid=grid,
        in_specs=[pl.BlockSpec((DIM, c), lambda i: (0, i)),
                  pl.BlockSpec((DIM, DIM), lambda i: (0, 0))],
        out_specs=pl.BlockSpec((c, 2 * DIM), lambda i: (i, 0)),
        out_shape=jax.ShapeDtypeStruct((n, 2 * DIM), jnp.float32),
    )(tab_t, eye)


def _bdot(a, b):
    return jnp.dot(a.astype(jnp.bfloat16), b.astype(jnp.bfloat16),
                   preferred_element_type=jnp.float32)


def _dense_body(um, im, w0u, w0i, b0, w1, b1, w2, b2, wpx, bp, out):
    x = _bdot(um[...], w0u[...]) + _bdot(im[...], w0i[...])
    x = jnp.maximum(x + b0[...], 0.0)
    x = jnp.maximum(_bdot(x, w1[...]) + b1[...], 0.0)
    x = jnp.maximum(_bdot(x, w2[...]) + b2[...], 0.0)
    out[...] = _bdot(x, wpx[...]) + bp[...]


def _final_body(ug, ig, xd, wpg, out):
    g = ug[:, :DIM] * ig[:, :DIM]
    out[...] = _bdot(g, wpg[...]) + xd[...]


_T = 2048


def _run_dense(um, im, W0, b0, W1, b1, W2, b2, Wp, bp):
    grid = (BATCH // _T,)
    f32 = jnp.float32
    w0u = W0[:, :DIM_MLP].T
    w0i = W0[:, DIM_MLP:].T
    w1 = W1.T
    w2 = W2.T
    wpx = Wp[:, DIM:].T
    b0r = b0.reshape(1, -1)
    b1r = b1.reshape(1, -1)
    b2r = b2.reshape(1, -1)
    bpr = bp.reshape(1, 1)

    batch_spec = lambda d: pl.BlockSpec((_T, d), lambda i: (i, 0))
    full_spec = lambda a, b: pl.BlockSpec((a, b), lambda i: (0, 0))

    return pl.pallas_call(
        _dense_body,
        grid=grid,
        in_specs=[
            batch_spec(DIM_MLP), batch_spec(DIM_MLP),
            full_spec(256, 256), full_spec(256, 256), full_spec(1, 256),
            full_spec(256, 128), full_spec(1, 128),
            full_spec(128, 64), full_spec(1, 64),
            full_spec(64, 1), full_spec(1, 1),
        ],
        out_specs=pl.BlockSpec((_T, 1), lambda i: (i, 0)),
        out_shape=jax.ShapeDtypeStruct((BATCH, 1), f32),
    )(um, im, w0u, w0i, b0r, w1, b1r, w2, b2r, wpx, bpr)


def _run_final(ug2, ig2, xdot, Wp):
    grid = (BATCH // _T,)
    f32 = jnp.float32
    wpg = Wp[:, :DIM].T
    batch_spec = lambda d: pl.BlockSpec((_T, d), lambda i: (i, 0))
    out = pl.pallas_call(
        _final_body,
        grid=grid,
        in_specs=[
            batch_spec(2 * DIM), batch_spec(2 * DIM), batch_spec(1),
            pl.BlockSpec((DIM, 1), lambda i: (0, 0)),
        ],
        out_specs=pl.BlockSpec((_T, 1), lambda i: (i, 0)),
        out_shape=jax.ShapeDtypeStruct((BATCH, 1), f32),
    )(ug2, ig2, xdot, wpg)
    return out.reshape(-1)


def kernel(userIdx, servIdx, U_gmf, U_mlp, I_gmf, I_mlp, W0, b0, W1, b1, W2, b2, Wp, bp):
    ui = userIdx.astype(jnp.int32)
    si = servIdx.astype(jnp.int32)
    um, im = _gather_mlp(ui, si, U_mlp, I_mlp)
    igp = _transpose_pad(I_gmf.T)
    ig2 = _gather_gmf1(si, igp)
    ugp = _transpose_pad(U_gmf.T)
    ug2 = _gather_gmf1(ui, ugp)
    xdot = _run_dense(um, im, W0, b0, W1, b1, W2, b2, Wp, bp)
    return _run_final(ug2, ig2, xdot, Wp)


# speedup vs baseline: 2.3369x; 1.0233x over previous
import functools

import jax
import jax.numpy as jnp
from jax import lax
from jax.experimental import pallas as pl
from jax.experimental.pallas import tpu as pltpu
from jax.experimental.pallas import tpu_sc as plsc

BATCH = 16384
DIM = 64
DIM_MLP = 256
CHUNK = 64


def _make_gather1(d):
    info = plsc.get_sparse_core_info()
    nc, ns = info.num_cores, info.num_subcores
    nw = nc * ns
    b_per_w = BATCH // nw
    n_chunks = b_per_w // CHUNK
    mesh = plsc.VectorSubcoreMesh(core_axis_name="c", subcore_axis_name="s")
    f32 = jnp.float32

    @functools.partial(
        pl.kernel,
        mesh=mesh,
        out_type=jax.ShapeDtypeStruct((BATCH, d), f32),
        scratch_types=[
            pltpu.VMEM((n_chunks, CHUNK), jnp.int32),
            pltpu.VMEM((CHUNK, d), f32),
            pltpu.VMEM((CHUNK, d), f32),
            pltpu.SemaphoreType.DMA,
            pltpu.SemaphoreType.DMA,
        ],
    )
    def gather1(idx_hbm, tab_hbm, out, idx, b0, b1, s0, s1):
        wid = lax.axis_index("s") * nc + lax.axis_index("c")
        base = wid * b_per_w
        bufs, sems = (b0, b1), (s0, s1)
        for j in range(n_chunks):
            pltpu.sync_copy(idx_hbm.at[pl.ds(base + j * CHUNK, CHUNK)],
                            idx.at[j])
        cps = [None, None]
        cps[0] = pltpu.async_copy(tab_hbm.at[idx.at[0]], bufs[0], sems[0])
        for j in range(n_chunks):
            s = j % 2
            n = (j + 1) % 2
            if j + 1 < n_chunks:
                cps[n] = pltpu.async_copy(tab_hbm.at[idx.at[j + 1]],
                                          bufs[n], sems[n])
            cps[s].wait()
            pltpu.sync_copy(bufs[s], out.at[pl.ds(base + j * CHUNK, CHUNK)])

    return gather1


def _make_gather(d):
    info = plsc.get_sparse_core_info()
    nc, ns = info.num_cores, info.num_subcores
    nw = nc * ns
    b_per_w = BATCH // nw
    n_chunks = b_per_w // CHUNK
    mesh = plsc.VectorSubcoreMesh(core_axis_name="c", subcore_axis_name="s")
    f32 = jnp.float32

    @functools.partial(
        pl.kernel,
        mesh=mesh,
        out_type=[
            jax.ShapeDtypeStruct((BATCH, d), f32),
            jax.ShapeDtypeStruct((BATCH, d), f32),
        ],
        scratch_types=[
            pltpu.VMEM((n_chunks, CHUNK), jnp.int32),
            pltpu.VMEM((n_chunks, CHUNK), jnp.int32),
            pltpu.VMEM((CHUNK, d), f32),
            pltpu.VMEM((CHUNK, d), f32),
            pltpu.VMEM((CHUNK, d), f32),
            pltpu.VMEM((CHUNK, d), f32),
            pltpu.SemaphoreType.DMA,
            pltpu.SemaphoreType.DMA,
            pltpu.SemaphoreType.DMA,
            pltpu.SemaphoreType.DMA,
        ],
    )
    def gather_kernel(u_idx_hbm, s_idx_hbm, ut_hbm, it_hbm,
                      out_u, out_i,
                      idx_u, idx_i, bu0, bu1, bi0, bi1,
                      su0, su1, si0, si1):
        wid = lax.axis_index("s") * nc + lax.axis_index("c")
        base = wid * b_per_w
        bufs_u, bufs_i = (bu0, bu1), (bi0, bi1)
        sems_u, sems_i = (su0, su1), (si0, si1)
        for j in range(n_chunks):
            pltpu.sync_copy(u_idx_hbm.at[pl.ds(base + j * CHUNK, CHUNK)],
                            idx_u.at[j])
            pltpu.sync_copy(s_idx_hbm.at[pl.ds(base + j * CHUNK, CHUNK)],
                            idx_i.at[j])

        cps = [None, None]
        cps[0] = (
            pltpu.async_copy(ut_hbm.at[idx_u.at[0]], bufs_u[0], sems_u[0]),
            pltpu.async_copy(it_hbm.at[idx_i.at[0]], bufs_i[0], sems_i[0]),
        )
        for j in range(n_chunks):
            s = j % 2
            n = (j + 1) % 2
            if j + 1 < n_chunks:
                cps[n] = (
                    pltpu.async_copy(ut_hbm.at[idx_u.at[j + 1]],
                                     bufs_u[n], sems_u[n]),
                    pltpu.async_copy(it_hbm.at[idx_i.at[j + 1]],
                                     bufs_i[n], sems_i[n]),
                )
            sl = pl.ds(base + j * CHUNK, CHUNK)
            cps[s][0].wait()
            pltpu.sync_copy(bufs_u[s], out_u.at[sl])
            cps[s][1].wait()
            pltpu.sync_copy(bufs_i[s], out_i.at[sl])

    return gather_kernel


_gather_mlp = _make_gather(DIM_MLP)
_gather_gmf1 = _make_gather1(2 * DIM)


def _tpad_body(tt, eye, out):
    t = jax.lax.dot_general(tt[...], eye[...], (((0,), (0,)), ((), ())),
                            preferred_element_type=jnp.float32)
    out[...] = jnp.pad(t, ((0, 0), (0, DIM)))


def _transpose_pad(tab_t):
    n = tab_t.shape[1]
    c = 16384
    grid = (-(-n // c),)
    eye = jnp.eye(DIM, dtype=jnp.float32)
    return pl.pallas_call(
        _tpad_body,
        grid=grid,
        in_specs=[pl.BlockSpec((DIM, c), lambda i: (0, i)),
                  pl.BlockSpec((DIM, DIM), lambda i: (0, 0))],
        out_specs=pl.BlockSpec((c, 2 * DIM), lambda i: (i, 0)),
        out_shape=jax.ShapeDtypeStruct((n, 2 * DIM), jnp.float32),
    )(tab_t, eye)


def _bdot(a, b):
    return jnp.dot(a.astype(jnp.bfloat16), b.astype(jnp.bfloat16),
                   preferred_element_type=jnp.float32)


def _dense_body(um, im, w0u, w0i, b0, w1, b1, w2, b2, wpx, bp, out):
    x = _bdot(um[...], w0u[...]) + _bdot(im[...], w0i[...])
    x = jnp.maximum(x + b0[...], 0.0)
    x = jnp.maximum(_bdot(x, w1[...]) + b1[...], 0.0)
    x = jnp.maximum(_bdot(x, w2[...]) + b2[...], 0.0)
    out[...] = _bdot(x, wpx[...]) + bp[...]


def _final_body(ug, ig, xd, wpg, out):
    g = ug[:, :DIM] * ig[:, :DIM]
    out[...] = _bdot(g, wpg[...]) + xd[...]


_T = 4096


def _run_dense(um, im, W0, b0, W1, b1, W2, b2, Wp, bp):
    grid = (BATCH // _T,)
    f32 = jnp.float32
    w0u = W0[:, :DIM_MLP].T
    w0i = W0[:, DIM_MLP:].T
    w1 = W1.T
    w2 = W2.T
    wpx = Wp[:, DIM:].T
    b0r = b0.reshape(1, -1)
    b1r = b1.reshape(1, -1)
    b2r = b2.reshape(1, -1)
    bpr = bp.reshape(1, 1)

    batch_spec = lambda d: pl.BlockSpec((_T, d), lambda i: (i, 0))
    full_spec = lambda a, b: pl.BlockSpec((a, b), lambda i: (0, 0))

    return pl.pallas_call(
        _dense_body,
        grid=grid,
        in_specs=[
            batch_spec(DIM_MLP), batch_spec(DIM_MLP),
            full_spec(256, 256), full_spec(256, 256), full_spec(1, 256),
            full_spec(256, 128), full_spec(1, 128),
            full_spec(128, 64), full_spec(1, 64),
            full_spec(64, 1), full_spec(1, 1),
        ],
        out_specs=pl.BlockSpec((_T, 1), lambda i: (i, 0)),
        out_shape=jax.ShapeDtypeStruct((BATCH, 1), f32),
    )(um, im, w0u, w0i, b0r, w1, b1r, w2, b2r, wpx, bpr)


def _run_final(ug2, ig2, xdot, Wp):
    grid = (BATCH // _T,)
    f32 = jnp.float32
    wpg = Wp[:, :DIM].T
    batch_spec = lambda d: pl.BlockSpec((_T, d), lambda i: (i, 0))
    out = pl.pallas_call(
        _final_body,
        grid=grid,
        in_specs=[
            batch_spec(2 * DIM), batch_spec(2 * DIM), batch_spec(1),
            pl.BlockSpec((DIM, 1), lambda i: (0, 0)),
        ],
        out_specs=pl.BlockSpec((_T, 1), lambda i: (i, 0)),
        out_shape=jax.ShapeDtypeStruct((BATCH, 1), f32),
    )(ug2, ig2, xdot, wpg)
    return out.reshape(-1)


def kernel(userIdx, servIdx, U_gmf, U_mlp, I_gmf, I_mlp, W0, b0, W1, b1, W2, b2, Wp, bp):
    ui = userIdx.astype(jnp.int32)
    si = servIdx.astype(jnp.int32)
    um, im = _gather_mlp(ui, si, U_mlp, I_mlp)
    igp = _transpose_pad(I_gmf.T)
    ig2 = _gather_gmf1(si, igp)
    ugp = _transpose_pad(U_gmf.T)
    ug2 = _gather_gmf1(ui, ugp)
    xdot = _run_dense(um, im, W0, b0, W1, b1, W2, b2, Wp, bp)
    return _run_final(ug2, ig2, xdot, Wp)


# speedup vs baseline: 2.4133x; 1.0327x over previous
import functools

import jax
import jax.numpy as jnp
from jax import lax
from jax.experimental import pallas as pl
from jax.experimental.pallas import tpu as pltpu
from jax.experimental.pallas import tpu_sc as plsc

BATCH = 16384
DIM = 64
DIM_MLP = 256
CHUNK = 64


def _make_gather1(d):
    info = plsc.get_sparse_core_info()
    nc, ns = info.num_cores, info.num_subcores
    nw = nc * ns
    b_per_w = BATCH // nw
    n_chunks = b_per_w // CHUNK
    mesh = plsc.VectorSubcoreMesh(core_axis_name="c", subcore_axis_name="s")
    f32 = jnp.float32

    @functools.partial(
        pl.kernel,
        mesh=mesh,
        out_type=jax.ShapeDtypeStruct((BATCH, d), f32),
        scratch_types=[
            pltpu.VMEM((n_chunks, CHUNK), jnp.int32),
            pltpu.VMEM((CHUNK, d), f32),
            pltpu.VMEM((CHUNK, d), f32),
            pltpu.SemaphoreType.DMA,
            pltpu.SemaphoreType.DMA,
        ],
    )
    def gather1(idx_hbm, tab_hbm, out, idx, b0, b1, s0, s1):
        wid = lax.axis_index("s") * nc + lax.axis_index("c")
        base = wid * b_per_w
        bufs, sems = (b0, b1), (s0, s1)
        for j in range(n_chunks):
            pltpu.sync_copy(idx_hbm.at[pl.ds(base + j * CHUNK, CHUNK)],
                            idx.at[j])
        cps = [None, None]
        cps[0] = pltpu.async_copy(tab_hbm.at[idx.at[0]], bufs[0], sems[0])
        for j in range(n_chunks):
            s = j % 2
            n = (j + 1) % 2
            if j + 1 < n_chunks:
                cps[n] = pltpu.async_copy(tab_hbm.at[idx.at[j + 1]],
                                          bufs[n], sems[n])
            cps[s].wait()
            pltpu.sync_copy(bufs[s], out.at[pl.ds(base + j * CHUNK, CHUNK)])

    return gather1


def _make_gather(d):
    info = plsc.get_sparse_core_info()
    nc, ns = info.num_cores, info.num_subcores
    nw = nc * ns
    b_per_w = BATCH // nw
    n_chunks = b_per_w // CHUNK
    mesh = plsc.VectorSubcoreMesh(core_axis_name="c", subcore_axis_name="s")
    f32 = jnp.float32

    @functools.partial(
        pl.kernel,
        mesh=mesh,
        out_type=[
            jax.ShapeDtypeStruct((BATCH, d), f32),
            jax.ShapeDtypeStruct((BATCH, d), f32),
        ],
        scratch_types=[
            pltpu.VMEM((n_chunks, CHUNK), jnp.int32),
            pltpu.VMEM((n_chunks, CHUNK), jnp.int32),
            pltpu.VMEM((CHUNK, d), f32),
            pltpu.VMEM((CHUNK, d), f32),
            pltpu.VMEM((CHUNK, d), f32),
            pltpu.VMEM((CHUNK, d), f32),
            pltpu.SemaphoreType.DMA,
            pltpu.SemaphoreType.DMA,
            pltpu.SemaphoreType.DMA,
            pltpu.SemaphoreType.DMA,
        ],
    )
    def gather_kernel(u_idx_hbm, s_idx_hbm, ut_hbm, it_hbm,
                      out_u, out_i,
                      idx_u, idx_i, bu0, bu1, bi0, bi1,
                      su0, su1, si0, si1):
        wid = lax.axis_index("s") * nc + lax.axis_index("c")
        base = wid * b_per_w
        bufs_u, bufs_i = (bu0, bu1), (bi0, bi1)
        sems_u, sems_i = (su0, su1), (si0, si1)
        for j in range(n_chunks):
            pltpu.sync_copy(u_idx_hbm.at[pl.ds(base + j * CHUNK, CHUNK)],
                            idx_u.at[j])
            pltpu.sync_copy(s_idx_hbm.at[pl.ds(base + j * CHUNK, CHUNK)],
                            idx_i.at[j])

        cps = [None, None]
        cps[0] = (
            pltpu.async_copy(ut_hbm.at[idx_u.at[0]], bufs_u[0], sems_u[0]),
            pltpu.async_copy(it_hbm.at[idx_i.at[0]], bufs_i[0], sems_i[0]),
        )
        for j in range(n_chunks):
            s = j % 2
            n = (j + 1) % 2
            if j + 1 < n_chunks:
                cps[n] = (
                    pltpu.async_copy(ut_hbm.at[idx_u.at[j + 1]],
                                     bufs_u[n], sems_u[n]),
                    pltpu.async_copy(it_hbm.at[idx_i.at[j + 1]],
                                     bufs_i[n], sems_i[n]),
                )
            sl = pl.ds(base + j * CHUNK, CHUNK)
            cps[s][0].wait()
            pltpu.sync_copy(bufs_u[s], out_u.at[sl])
            cps[s][1].wait()
            pltpu.sync_copy(bufs_i[s], out_i.at[sl])

    return gather_kernel


_gather_mlp = _make_gather(DIM_MLP)
_gather_gmf1 = _make_gather1(2 * DIM)


def _tpad_body(tt, eye, out):
    t = jax.lax.dot_general(tt[...], eye[...], (((0,), (0,)), ((), ())),
                            preferred_element_type=jnp.float32)
    out[...] = jnp.pad(t, ((0, 0), (0, DIM)))


def _transpose_pad(tab_t):
    n = tab_t.shape[1]
    c = 16384
    grid = (-(-n // c),)
    eye = jnp.eye(DIM, dtype=jnp.float32)
    return pl.pallas_call(
        _tpad_body,
        grid=grid,
        in_specs=[pl.BlockSpec((DIM, c), lambda i: (0, i)),
                  pl.BlockSpec((DIM, DIM), lambda i: (0, 0))],
        out_specs=pl.BlockSpec((c, 2 * DIM), lambda i: (i, 0)),
        out_shape=jax.ShapeDtypeStruct((n, 2 * DIM), jnp.float32),
    )(tab_t, eye)


def _bdot(a, b):
    return jnp.dot(a.astype(jnp.bfloat16), b.astype(jnp.bfloat16),
                   preferred_element_type=jnp.float32)


def _dense_body(um, im, w0u, w0i, b0, w1, b1, w2, b2, wpx, bp, out):
    x = _bdot(um[...], w0u[...]) + _bdot(im[...], w0i[...])
    x = jnp.maximum(x + b0[...], 0.0)
    x = jnp.maximum(_bdot(x, w1[...]) + b1[...], 0.0)
    x = jnp.maximum(_bdot(x, w2[...]) + b2[...], 0.0)
    out[...] = _bdot(x, wpx[...]) + bp[...]


def _final_body(ug, ig, xd, wpg, out):
    g = ug[:, :DIM] * ig[:, :DIM]
    out[...] = _bdot(g, wpg[...]) + xd[...]


_T = 4096


def _run_dense(um, im, W0, b0, W1, b1, W2, b2, Wp, bp):
    grid = (BATCH // _T,)
    f32 = jnp.float32
    w0u = W0[:, :DIM_MLP].T
    w0i = W0[:, DIM_MLP:].T
    w1 = W1.T
    w2 = W2.T
    wpx = Wp[:, DIM:].T
    b0r = b0.reshape(1, -1)
    b1r = b1.reshape(1, -1)
    b2r = b2.reshape(1, -1)
    bpr = bp.reshape(1, 1)

    batch_spec = lambda d: pl.BlockSpec((_T, d), lambda i: (i, 0))
    full_spec = lambda a, b: pl.BlockSpec((a, b), lambda i: (0, 0))

    return pl.pallas_call(
        _dense_body,
        grid=grid,
        in_specs=[
            batch_spec(DIM_MLP), batch_spec(DIM_MLP),
            full_spec(256, 256), full_spec(256, 256), full_spec(1, 256),
            full_spec(256, 128), full_spec(1, 128),
            full_spec(128, 64), full_spec(1, 64),
            full_spec(64, 1), full_spec(1, 1),
        ],
        out_specs=pl.BlockSpec((_T, 1), lambda i: (i, 0)),
        out_shape=jax.ShapeDtypeStruct((BATCH, 1), f32),
    )(um, im, w0u, w0i, b0r, w1, b1r, w2, b2r, wpx, bpr)


def _run_final(ug2, ig2, xdot, Wp):
    grid = (BATCH // _T,)
    f32 = jnp.float32
    wpg = Wp[:, :DIM].T
    batch_spec = lambda d: pl.BlockSpec((_T, d), lambda i: (i, 0))
    out = pl.pallas_call(
        _final_body,
        grid=grid,
        in_specs=[
            batch_spec(2 * DIM), batch_spec(2 * DIM), batch_spec(1),
            pl.BlockSpec((DIM, 1), lambda i: (0, 0)),
        ],
        out_specs=pl.BlockSpec((_T, 1), lambda i: (i, 0)),
        out_shape=jax.ShapeDtypeStruct((BATCH, 1), f32),
    )(ug2, ig2, xdot, wpg)
    return out.reshape(-1)


def kernel(userIdx, servIdx, U_gmf, U_mlp, I_gmf, I_mlp, W0, b0, W1, b1, W2, b2, Wp, bp):
    ui = userIdx.astype(jnp.int32)
    si = servIdx.astype(jnp.int32)
    um, im = _gather_mlp(ui, si, U_mlp, I_mlp)
    igp = _transpose_pad(I_gmf.T)
    ig2 = _gather_gmf1(si, igp)
    ugt, igp = jax.lax.optimization_barrier((U_gmf.T, igp))
    ugp = _transpose_pad(ugt)
    ug2 = _gather_gmf1(ui, ugp)
    xdot = _run_dense(um, im, W0, b0, W1, b1, W2, b2, Wp, bp)
    return _run_final(ug2, ig2, xdot, Wp)
